# Initial kernel scaffold; baseline (speedup 1.0000x reference)
import jax
import jax.numpy as jnp
from jax.experimental import pallas as pl


def kernel(x, length, emb, W, b):
    raise NotImplementedError("write your pallas kernel here")


# speedup vs baseline: 1.2055x; 1.2055x over previous
import functools

import jax
import jax.numpy as jnp
from jax import lax
from jax.experimental import pallas as pl
from jax.experimental.pallas import tpu as pltpu
from jax.experimental.pallas import tpu_sc as plsc

VOCAB = 100000
EMB = 300
EMBP = 304
HID = 128
B = 4096
L = 50

NC = 2
NS = 16
NW = NC * NS
BAGS_PER_W = B // NW
CHUNK_BAGS = 2
ROWS = CHUNK_BAGS * L
NCHUNKS = BAGS_PER_W // CHUNK_BAGS

_mesh = plsc.VectorSubcoreMesh(core_axis_name="c", subcore_axis_name="s")


@functools.partial(
    pl.kernel,
    mesh=_mesh,
    out_type=jax.ShapeDtypeStruct((B, EMBP), jnp.float32),
    compiler_params=pltpu.CompilerParams(use_tc_tiling_on_sc=False),
    scratch_types=[
        pltpu.VMEM((NCHUNKS, ROWS), jnp.int32),
        pltpu.VMEM((NCHUNKS, ROWS), jnp.int32),
        pltpu.VMEM((ROWS, EMBP), jnp.float32),
        pltpu.VMEM_SHARED((NS * BAGS_PER_W, EMBP), jnp.float32),
    ],
)
def _sc_pool(x_hbm, emb_hbm, bagmap_hbm, zeros_hbm, out_hbm,
             idx_v, bagmap_v, rows_v, acc_sh):
    sid = lax.axis_index("s")
    wid = sid * NC + lax.axis_index("c")
    base = wid * BAGS_PER_W
    pltpu.sync_copy(x_hbm.at[wid], idx_v)
    pltpu.sync_copy(bagmap_hbm.at[sid], bagmap_v)
    pltpu.sync_copy(zeros_hbm, acc_sh.at[pl.ds(sid * BAGS_PER_W, BAGS_PER_W)])

    @pl.loop(0, NCHUNKS)
    def _(c):
        pltpu.sync_copy(emb_hbm.at[idx_v.at[c]], rows_v)
        pltpu.sync_copy(rows_v, acc_sh.at[bagmap_v.at[c]], add=True)

    pltpu.sync_copy(
        acc_sh.at[pl.ds(sid * BAGS_PER_W, BAGS_PER_W)],
        out_hbm.at[pl.ds(base, BAGS_PER_W)],
    )


_TC_BLK = 512


def _proj_body(s_ref, len_ref, wt_ref, b_ref, o_ref):
    s = s_ref[...] / len_ref[...]
    o_ref[...] = (
        jnp.dot(s, wt_ref[...], preferred_element_type=jnp.float32) + b_ref[...]
    )


def _tc_project(sums, length_f, Wt, b2):
    return pl.pallas_call(
        _proj_body,
        grid=(B // _TC_BLK,),
        in_specs=[
            pl.BlockSpec((_TC_BLK, EMBP), lambda i: (i, 0)),
            pl.BlockSpec((_TC_BLK, 1), lambda i: (i, 0)),
            pl.BlockSpec((EMBP, HID), lambda i: (0, 0)),
            pl.BlockSpec((1, HID), lambda i: (0, 0)),
        ],
        out_specs=pl.BlockSpec((_TC_BLK, HID), lambda i: (i, 0)),
        out_shape=jax.ShapeDtypeStruct((B, HID), jnp.float32),
    )(sums, length_f, Wt, b2)


@jax.jit
def kernel(x, length, emb, W, b):
    x3d = x.astype(jnp.int32).reshape(NW, NCHUNKS, ROWS)
    embp = jnp.pad(emb, ((0, 0), (0, EMBP - EMB)))
    bagmap = jnp.repeat(
        jnp.arange(NS * BAGS_PER_W, dtype=jnp.int32), L
    ).reshape(NS, NCHUNKS, ROWS)
    zeros = jnp.zeros((BAGS_PER_W, EMBP), jnp.float32)
    sums = _sc_pool(x3d, embp, bagmap, zeros)
    length_f = length.astype(jnp.float32).reshape(B, 1)
    Wtp = jnp.pad(W.T, ((0, EMBP - EMB), (0, 0)))
    return _tc_project(sums, length_f, Wtp, b.reshape(1, HID))
